# initial kernel scaffold (unmeasured)
import jax
import jax.numpy as jnp
from jax import lax
from jax.experimental import pallas as pl
from jax.experimental.pallas import tpu as pltpu

N_DEV = 8


def _gelu(y):
    c = 0.7978845608028654
    return 0.5 * y * (1.0 + jnp.tanh(c * (y + 0.044715 * y * y * y)))


def kernel(x, w_mat):
    m_per, k = x.shape
    _, n_per = w_mat.shape

    def body(x_ref, w_ref, out_ref, comm_ref, send_sems, recv_sems):
        my = lax.axis_index("i")
        left = lax.rem(my + (N_DEV - 1), N_DEV)
        right = lax.rem(my + 1, N_DEV)

        barrier_sem = pltpu.get_barrier_semaphore()
        pl.semaphore_signal(barrier_sem, inc=1, device_id=(left,),
                            device_id_type=pl.DeviceIdType.MESH)
        pl.semaphore_signal(barrier_sem, inc=1, device_id=(right,),
                            device_id_type=pl.DeviceIdType.MESH)
        pl.semaphore_wait(barrier_sem, 2)

        comm_ref[0] = x_ref[...]
        y = jnp.dot(x_ref[...], w_ref[...], preferred_element_type=jnp.float32)
        out_ref[pl.ds(my * m_per, m_per), :] = _gelu(y)

        for h in range(N_DEV - 1):
            s = h % 2
            r = (h + 1) % 2
            rdma = pltpu.make_async_remote_copy(
                src_ref=comm_ref.at[s],
                dst_ref=comm_ref.at[r],
                send_sem=send_sems.at[s],
                recv_sem=recv_sems.at[r],
                device_id=(right,),
                device_id_type=pl.DeviceIdType.MESH,
            )
            rdma.start()
            rdma.wait()
            origin = lax.rem(my + (N_DEV - 1 - h) + N_DEV - 2, N_DEV) * 0 + \
                lax.rem(my - h - 1 + N_DEV, N_DEV)
            y = jnp.dot(comm_ref[r], w_ref[...],
                        preferred_element_type=jnp.float32)
            out_ref[pl.ds(origin * m_per, m_per), :] = _gelu(y)

    return pl.pallas_call(
        body,
        out_shape=jax.ShapeDtypeStruct((N_DEV * m_per, n_per), jnp.float32),
        in_specs=[
            pl.BlockSpec(memory_space=pltpu.VMEM),
            pl.BlockSpec(memory_space=pltpu.VMEM),
        ],
        out_specs=pl.BlockSpec(memory_space=pltpu.VMEM),
        scratch_shapes=[
            pltpu.VMEM((2, m_per, k), jnp.float32),
            pltpu.SemaphoreType.DMA((2,)),
            pltpu.SemaphoreType.DMA((2,)),
        ],
        compiler_params=pltpu.CompilerParams(collective_id=0),
    )(x, w_mat)


# baseline (device time: 715025 ns/iter reference)
import jax
import jax.numpy as jnp
from jax import lax
from jax.experimental import pallas as pl
from jax.experimental.pallas import tpu as pltpu

N_DEV = 8


def _gelu(y):
    c = 0.7978845608028654
    return 0.5 * y * (1.0 + jnp.tanh(c * (y + 0.044715 * y * y * y)))


def kernel(x, w_mat):
    m_per, k = x.shape
    _, n_per = w_mat.shape

    def body(x_ref, w_ref, out_ref, comm_ref, send_sems, recv_sems):
        my = lax.axis_index("i")
        left = lax.rem(my + (N_DEV - 1), N_DEV)
        right = lax.rem(my + 1, N_DEV)

        barrier_sem = pltpu.get_barrier_semaphore()
        pl.semaphore_signal(barrier_sem, inc=1, device_id=(left,),
                            device_id_type=pl.DeviceIdType.MESH)
        pl.semaphore_signal(barrier_sem, inc=1, device_id=(right,),
                            device_id_type=pl.DeviceIdType.MESH)
        pl.semaphore_wait(barrier_sem, 2)

        comm_ref[0] = x_ref[...]
        y = jnp.dot(x_ref[...], w_ref[...], preferred_element_type=jnp.float32)
        out_ref[pl.ds(my * m_per, m_per), :] = _gelu(y)

        for h in range(N_DEV - 1):
            s = h % 2
            r = (h + 1) % 2
            rdma = pltpu.make_async_remote_copy(
                src_ref=comm_ref.at[s],
                dst_ref=comm_ref.at[r],
                send_sem=send_sems.at[s],
                recv_sem=recv_sems.at[r],
                device_id=(right,),
                device_id_type=pl.DeviceIdType.MESH,
            )
            rdma.start()
            rdma.wait()
            origin = lax.rem(my - h - 1 + N_DEV, N_DEV)
            y = jnp.dot(comm_ref[r], w_ref[...],
                        preferred_element_type=jnp.float32)
            out_ref[pl.ds(origin * m_per, m_per), :] = _gelu(y)

    return pl.pallas_call(
        body,
        out_shape=jax.ShapeDtypeStruct((N_DEV * m_per, n_per), jnp.float32),
        in_specs=[
            pl.BlockSpec(memory_space=pltpu.VMEM),
            pl.BlockSpec(memory_space=pltpu.VMEM),
        ],
        out_specs=pl.BlockSpec(memory_space=pltpu.VMEM),
        scratch_shapes=[
            pltpu.VMEM((2, m_per, k), jnp.float32),
            pltpu.SemaphoreType.DMA((2,)),
            pltpu.SemaphoreType.DMA((2,)),
        ],
        compiler_params=pltpu.CompilerParams(
            collective_id=0,
            vmem_limit_bytes=100 * 1024 * 1024,
        ),
    )(x, w_mat)


# device time: 374439 ns/iter; 1.9096x vs baseline; 1.9096x over previous
import jax
import jax.numpy as jnp
from jax import lax
from jax.experimental import pallas as pl
from jax.experimental.pallas import tpu as pltpu

N_DEV = 8
_MESH = pl.DeviceIdType.MESH


def _gelu(y):
    c = 0.7978845608028654
    return 0.5 * y * (1.0 + jnp.tanh(c * (y + 0.044715 * y * y * y)))


def kernel(x, w_mat):
    m_per, k = x.shape
    _, n_per = w_mat.shape
    half = m_per // 2

    def body(x_ref, w_ref, out_ref, cw_ref, ccw_ref,
             cw_send, cw_recv, ccw_send, ccw_recv, cw_credit, ccw_credit):
        my = lax.axis_index("i")
        left = lax.rem(my + N_DEV - 1, N_DEV)
        right = lax.rem(my + 1, N_DEV)

        barrier_sem = pltpu.get_barrier_semaphore()
        pl.semaphore_signal(barrier_sem, inc=1, device_id=(left,),
                            device_id_type=_MESH)
        pl.semaphore_signal(barrier_sem, inc=1, device_id=(right,),
                            device_id_type=_MESH)
        pl.semaphore_wait(barrier_sem, 2)

        cw_ref[0] = x_ref[:half]
        ccw_ref[0] = x_ref[half:]

        def compute(slot, h):
            top_origin = lax.rem(my - h + N_DEV, N_DEV)
            bot_origin = lax.rem(my + h, N_DEV)
            yt = jnp.dot(cw_ref[slot], w_ref[...],
                         preferred_element_type=jnp.float32)
            out_ref[pl.ds(top_origin * m_per, half), :] = _gelu(yt)
            yb = jnp.dot(ccw_ref[slot], w_ref[...],
                         preferred_element_type=jnp.float32)
            out_ref[pl.ds(bot_origin * m_per + half, half), :] = _gelu(yb)

        for h in range(N_DEV - 1):
            s = h % 2
            r = (h + 1) % 2
            if h >= 1:
                pl.semaphore_wait(cw_credit, 1)
                pl.semaphore_wait(ccw_credit, 1)
            cw_rdma = pltpu.make_async_remote_copy(
                src_ref=cw_ref.at[s], dst_ref=cw_ref.at[r],
                send_sem=cw_send.at[s], recv_sem=cw_recv.at[r],
                device_id=(right,), device_id_type=_MESH,
            )
            ccw_rdma = pltpu.make_async_remote_copy(
                src_ref=ccw_ref.at[s], dst_ref=ccw_ref.at[r],
                send_sem=ccw_send.at[s], recv_sem=ccw_recv.at[r],
                device_id=(left,), device_id_type=_MESH,
            )
            cw_rdma.start()
            ccw_rdma.start()
            compute(s, h)
            cw_rdma.wait()
            ccw_rdma.wait()
            if h < N_DEV - 2:
                pl.semaphore_signal(cw_credit, inc=1, device_id=(left,),
                                    device_id_type=_MESH)
                pl.semaphore_signal(ccw_credit, inc=1, device_id=(right,),
                                    device_id_type=_MESH)

        compute((N_DEV - 1) % 2, N_DEV - 1)

    return pl.pallas_call(
        body,
        out_shape=jax.ShapeDtypeStruct((N_DEV * m_per, n_per), jnp.float32),
        in_specs=[
            pl.BlockSpec(memory_space=pltpu.VMEM),
            pl.BlockSpec(memory_space=pltpu.VMEM),
        ],
        out_specs=pl.BlockSpec(memory_space=pltpu.VMEM),
        scratch_shapes=[
            pltpu.VMEM((2, half, k), jnp.float32),
            pltpu.VMEM((2, half, k), jnp.float32),
            pltpu.SemaphoreType.DMA((2,)),
            pltpu.SemaphoreType.DMA((2,)),
            pltpu.SemaphoreType.DMA((2,)),
            pltpu.SemaphoreType.DMA((2,)),
            pltpu.SemaphoreType.REGULAR,
            pltpu.SemaphoreType.REGULAR,
        ],
        compiler_params=pltpu.CompilerParams(
            collective_id=0,
            vmem_limit_bytes=100 * 1024 * 1024,
        ),
    )(x, w_mat)


# device time: 351807 ns/iter; 2.0324x vs baseline; 1.0643x over previous
import jax
import jax.numpy as jnp
from jax import lax
from jax.experimental import pallas as pl
from jax.experimental.pallas import tpu as pltpu

N_DEV = 8
NS = 5
NOUT = 2 * (N_DEV - 1)
NITER = NOUT + 2
_MESH = pl.DeviceIdType.MESH


def _gelu(y):
    c = 0.7978845608028654
    return 0.5 * y * (1.0 + jnp.tanh(c * (y + 0.044715 * y * y * y)))


def kernel(x, w_mat):
    m_per, k = x.shape
    _, n_per = w_mat.shape
    q = m_per // 4

    def body(x_ref, w_ref, out_ref, cw_buf, ccw_buf,
             cw_ssem, cw_rsem, ccw_ssem, ccw_rsem, cw_credit, ccw_credit):
        my = lax.axis_index("i")
        left = lax.rem(my + N_DEV - 1, N_DEV)
        right = lax.rem(my + 1, N_DEV)

        barrier_sem = pltpu.get_barrier_semaphore()
        pl.semaphore_signal(barrier_sem, inc=1, device_id=(left,),
                            device_id_type=_MESH)
        pl.semaphore_signal(barrier_sem, inc=1, device_id=(right,),
                            device_id_type=_MESH)
        pl.semaphore_wait(barrier_sem, 2)

        cw_buf[0] = x_ref[0:q]
        cw_buf[1] = x_ref[q:2 * q]
        ccw_buf[0] = x_ref[2 * q:3 * q]
        ccw_buf[1] = x_ref[3 * q:4 * q]

        def gemm(j):
            cw_org = lax.rem(my - (j // 2) + N_DEV, N_DEV)
            yt = jnp.dot(cw_buf[j % NS], w_ref[...],
                         preferred_element_type=jnp.float32)
            out_ref[pl.ds(cw_org * m_per + (j % 2) * q, q), :] = _gelu(yt)
            ccw_org = lax.rem(my + (j // 2), N_DEV)
            yb = jnp.dot(ccw_buf[j % NS], w_ref[...],
                         preferred_element_type=jnp.float32)
            out_ref[pl.ds(ccw_org * m_per + 2 * q + (j % 2) * q, q), :] = \
                _gelu(yb)

        cw_desc = {}
        ccw_desc = {}
        for j in range(NITER):
            if j >= 2:
                cw_desc[j - 2].wait_recv()
                ccw_desc[j - 2].wait_recv()
            if j < NOUT:
                if j >= NS - 2:
                    pl.semaphore_wait(cw_credit, 1)
                    pl.semaphore_wait(ccw_credit, 1)
                cw_desc[j] = pltpu.make_async_remote_copy(
                    src_ref=cw_buf.at[j % NS],
                    dst_ref=cw_buf.at[(j + 2) % NS],
                    send_sem=cw_ssem.at[j % NS],
                    recv_sem=cw_rsem.at[(j + 2) % NS],
                    device_id=(right,), device_id_type=_MESH,
                )
                ccw_desc[j] = pltpu.make_async_remote_copy(
                    src_ref=ccw_buf.at[j % NS],
                    dst_ref=ccw_buf.at[(j + 2) % NS],
                    send_sem=ccw_ssem.at[j % NS],
                    recv_sem=ccw_rsem.at[(j + 2) % NS],
                    device_id=(left,), device_id_type=_MESH,
                )
                cw_desc[j].start()
                ccw_desc[j].start()
            gemm(j)
            if j >= 2:
                cw_desc[j - 2].wait_send()
                ccw_desc[j - 2].wait_send()
            if 2 <= j <= NOUT - NS + 3:
                pl.semaphore_signal(cw_credit, inc=1, device_id=(left,),
                                    device_id_type=_MESH)
                pl.semaphore_signal(ccw_credit, inc=1, device_id=(right,),
                                    device_id_type=_MESH)

    return pl.pallas_call(
        body,
        out_shape=jax.ShapeDtypeStruct((N_DEV * m_per, n_per), jnp.float32),
        in_specs=[
            pl.BlockSpec(memory_space=pltpu.VMEM),
            pl.BlockSpec(memory_space=pltpu.VMEM),
        ],
        out_specs=pl.BlockSpec(memory_space=pltpu.VMEM),
        scratch_shapes=[
            pltpu.VMEM((NS, q, k), jnp.float32),
            pltpu.VMEM((NS, q, k), jnp.float32),
            pltpu.SemaphoreType.DMA((NS,)),
            pltpu.SemaphoreType.DMA((NS,)),
            pltpu.SemaphoreType.DMA((NS,)),
            pltpu.SemaphoreType.DMA((NS,)),
            pltpu.SemaphoreType.REGULAR,
            pltpu.SemaphoreType.REGULAR,
        ],
        compiler_params=pltpu.CompilerParams(
            collective_id=0,
            vmem_limit_bytes=100 * 1024 * 1024,
        ),
    )(x, w_mat)
